# baseline (speedup 1.0000x reference)
import functools

import jax
import jax.numpy as jnp
from jax import lax
from jax.experimental import pallas as pl
from jax.experimental.pallas import tpu as pltpu
from jax.experimental.pallas import tpu_sc as plsc

N = 10000
NPAD = 10240
E = 160000
D = 300
DC = 80
NCH = 4
NSUB = 16
CHUNK = 80
EDGES_PER_SUB = E // NSUB
NCHUNK = EDGES_PER_SUB // CHUNK
ROWS_PER_SUB = NPAD // NSUB
ZROWS = 128
BLK = 1000
GRID = N // BLK


def _sc_mesh():
    return plsc.VectorSubcoreMesh(
        core_axis_name="c", subcore_axis_name="s", num_cores=2, num_subcores=NSUB
    )


def _stage_idx(idx_all, base, cur):
    for j in range(CHUNK // 16):
        cur[pl.ds(j * 16, 16)] = idx_all[pl.ds(base + j * 16, 16)]


NBUF = 6


def _segsum_body(x0, x2, ei_hbm, o0, o2,
                 srcv, dstv, curs, rows, zbuf, acc, gsems, ssems):
    core = lax.axis_index("c")
    w = lax.axis_index("s")

    @pl.loop(0, ZROWS)
    def _(i):
        for j in range(DC // 16):
            zbuf[i, pl.ds(j * 16, 16)] = jnp.zeros((16,), jnp.float32)

    pltpu.sync_copy(ei_hbm.at[0, pl.ds(w * EDGES_PER_SUB, EDGES_PER_SUB)], srcv)
    pltpu.sync_copy(ei_hbm.at[1, pl.ds(w * EDGES_PER_SUB, EDGES_PER_SUB)], dstv)

    def run(x_hbm, out_hbm):
        for k in range(ROWS_PER_SUB // ZROWS):
            pltpu.sync_copy(zbuf,
                            acc.at[pl.ds(w * ROWS_PER_SUB + k * ZROWS, ZROWS)])
        plsc.subcore_barrier()

        def g_issue(c, b):
            pltpu.async_copy(x_hbm.at[srcv.at[pl.ds(c * CHUNK, CHUNK)]],
                             rows[b], gsems[b])

        def g_wait(c, b):
            pltpu.make_async_copy(x_hbm.at[srcv.at[pl.ds(c * CHUNK, CHUNK)]],
                                  rows[b], gsems[b]).wait()

        def s_wait(b):
            pltpu.make_async_copy(rows[b], acc.at[curs[b]], ssems[b]).wait()

        for b in range(NBUF):
            g_issue(b, b)

        tail = NCHUNK % NBUF
        main = NCHUNK - tail

        @pl.loop(0, main, step=NBUF)
        def _(i):
            for b in range(NBUF):
                g_wait(i + b, b)
                _stage_idx(dstv, (i + b) * CHUNK, curs[b])
                pltpu.async_copy(rows[b], acc.at[curs[b]], ssems[b], add=True)
            for b in range(NBUF):
                @pl.when(i + b + NBUF < NCHUNK)
                def _():
                    s_wait(b)
                    g_issue(i + b + NBUF, b)

        for b in range(tail):
            g_wait(main + b, b)
            _stage_idx(dstv, (main + b) * CHUNK, curs[b])
            pltpu.async_copy(rows[b], acc.at[curs[b]], ssems[b], add=True)
        for b in range(NBUF):
            s_wait(b)

        plsc.subcore_barrier()
        pltpu.sync_copy(acc.at[pl.ds(w * ROWS_PER_SUB, ROWS_PER_SUB)],
                        out_hbm.at[pl.ds(w * ROWS_PER_SUB, ROWS_PER_SUB)])

    @pl.when(core == 0)
    def _():
        run(x0, o0)

    @pl.when(core == 1)
    def _():
        run(x2, o2)


@jax.jit
def _segsum_pass(xa, xb, edge_index):
    chunk_ty = jax.ShapeDtypeStruct((NPAD, DC), jnp.float32)
    f = pl.kernel(
        _segsum_body,
        out_type=(chunk_ty,) * 2,
        mesh=_sc_mesh(),
        scratch_types=[
            pltpu.VMEM((EDGES_PER_SUB,), jnp.int32),
            pltpu.VMEM((EDGES_PER_SUB,), jnp.int32),
            [pltpu.VMEM((CHUNK,), jnp.int32)] * NBUF,
            [pltpu.VMEM((CHUNK, DC), jnp.float32)] * NBUF,
            pltpu.VMEM((ZROWS, DC), jnp.float32),
            pltpu.VMEM_SHARED((NPAD, DC), jnp.float32),
            [pltpu.SemaphoreType.DMA] * NBUF,
            [pltpu.SemaphoreType.DMA] * NBUF,
        ],
        compiler_params=pltpu.CompilerParams(use_tc_tiling_on_sc=False),
        cost_estimate=pl.CostEstimate(
            flops=8 * E * DC, bytes_accessed=2 * (E + N) * DC * 4,
            transcendentals=0),
    )
    return f(xa, xb, edge_index)


def _split_body(x, o0, o1, o2, o3):
    xb = x[...]
    o0[...] = xb[:, 0:DC]
    o1[...] = xb[:, DC:2 * DC]
    o2[...] = xb[:, 2 * DC:3 * DC]
    o3[...] = jnp.concatenate(
        [xb[:, 3 * DC:D], jnp.zeros((BLK, NCH * DC - D), jnp.float32)], axis=1)


@jax.jit
def _tc_split(features):
    cspec = pl.BlockSpec((BLK, DC), lambda i: (i, 0))
    chunk_ty = jax.ShapeDtypeStruct((NPAD, DC), jnp.float32)
    return pl.pallas_call(
        _split_body,
        grid=(GRID,),
        in_specs=[pl.BlockSpec((BLK, D), lambda i: (i, 0))],
        out_specs=(cspec,) * NCH,
        out_shape=(chunk_ty,) * NCH,
    )(features)


def _pre_body(x0, x1, x2, x3, w0, w1, w2, w3, b0, b1, b2, b3,
              p0, p1, p2, p3):
    u = jnp.concatenate([x0[...], x1[...], x2[...], x3[...]], axis=1)
    for wq, bq, pq in ((w0, b0, p0), (w1, b1, p1), (w2, b2, p2), (w3, b3, p3)):
        pq[...] = jnp.dot(u, wq[...], preferred_element_type=jnp.float32) \
            + bq[...]


@jax.jit
def _tc_pre(x_chunks, wx_chunks, b_chunks):
    cspec = pl.BlockSpec((BLK, DC), lambda i: (i, 0))
    wspec = pl.BlockSpec((4 * DC, DC), lambda i: (0, 0))
    bspec = pl.BlockSpec((1, DC), lambda i: (0, 0))
    chunk_ty = jax.ShapeDtypeStruct((NPAD, DC), jnp.float32)
    return pl.pallas_call(
        _pre_body,
        grid=(GRID,),
        in_specs=[cspec] * 4 + [wspec] * 4 + [bspec] * 4,
        out_specs=(cspec,) * 4,
        out_shape=(chunk_ty,) * 4,
    )(*x_chunks, *wx_chunks, *b_chunks)


def _post_partial_body(p0, p1, p2, p3, aA, aB, w0, w1, w2, w3,
                       s0, s1, s2, s3):
    u = jnp.concatenate([aA[...], aB[...]], axis=1)
    for wq, pq, sq in ((w0, p0, s0), (w1, p1, s1), (w2, p2, s2), (w3, p3, s3)):
        sq[...] = pq[...] + jnp.dot(u, wq[...],
                                    preferred_element_type=jnp.float32)


@jax.jit
def _tc_post_partial(p_chunks, aA, aB, w_chunks):
    cspec = pl.BlockSpec((BLK, DC), lambda i: (i, 0))
    wspec = pl.BlockSpec((2 * DC, DC), lambda i: (0, 0))
    chunk_ty = jax.ShapeDtypeStruct((NPAD, DC), jnp.float32)
    return pl.pallas_call(
        _post_partial_body,
        grid=(GRID,),
        in_specs=[cspec] * 6 + [wspec] * 4,
        out_specs=(cspec,) * 4,
        out_shape=(chunk_ty,) * 4,
        cost_estimate=pl.CostEstimate(
            flops=2 * N * 2 * DC * 4 * DC, bytes_accessed=11 * NPAD * DC * 4,
            transcendentals=0),
    )(*p_chunks, aA, aB, *w_chunks)


def _post1b_body(s0, s1, s2, s3, aA, aB, w0, w1, w2, w3,
                 h0, h1, h2, h3):
    u = jnp.concatenate([aA[...], aB[...]], axis=1)
    for wq, sq, hq in ((w0, s0, h0), (w1, s1, h1), (w2, s2, h2), (w3, s3, h3)):
        z = sq[...] + jnp.dot(u, wq[...], preferred_element_type=jnp.float32)
        hq[...] = jnp.where(z > 0, z, 0.01 * z)


@jax.jit
def _tc_post1b(s_chunks, aA, aB, w_chunks):
    cspec = pl.BlockSpec((BLK, DC), lambda i: (i, 0))
    wspec = pl.BlockSpec((2 * DC, DC), lambda i: (0, 0))
    chunk_ty = jax.ShapeDtypeStruct((NPAD, DC), jnp.float32)
    return pl.pallas_call(
        _post1b_body,
        grid=(GRID,),
        in_specs=[cspec] * 6 + [wspec] * 4,
        out_specs=(cspec,) * 4,
        out_shape=(chunk_ty,) * 4,
    )(*s_chunks, aA, aB, *w_chunks)


def _finalB_body(s0, s1, s2, s3, aA, aB, w0, w1, w2, w3,
                 v0, v1, v2, v3, b3f, t_out, ssq):
    i = pl.program_id(0)
    u = jnp.concatenate([aA[...], aB[...]], axis=1)
    acc = b3f[...]
    for wq, sq, vq in ((w0, s0, v0), (w1, s1, v1), (w2, s2, v2), (w3, s3, v3)):
        z = sq[...] + jnp.dot(u, wq[...], preferred_element_type=jnp.float32)
        acc = acc + jnp.dot(z, vq[...], preferred_element_type=jnp.float32)
    t = jnp.tanh(acc)
    t_out[...] = t

    @pl.when(i == 0)
    def _():
        ssq[0, 0] = 0.0

    ssq[0, 0] += jnp.sum(t * t)


@jax.jit
def _tc_finalB(s_chunks, aA, aB, w_chunks, v_chunks, b3f):
    cspec = pl.BlockSpec((BLK, DC), lambda i: (i, 0))
    wspec = pl.BlockSpec((2 * DC, DC), lambda i: (0, 0))
    vspec = pl.BlockSpec((DC, D), lambda i: (0, 0))
    b3spec = pl.BlockSpec((1, D), lambda i: (0, 0))
    return pl.pallas_call(
        _finalB_body,
        grid=(GRID,),
        in_specs=[cspec] * 6 + [wspec] * 4 + [vspec] * 4 + [b3spec],
        out_specs=(
            pl.BlockSpec((BLK, D), lambda i: (i, 0)),
            pl.BlockSpec((1, 1), lambda i: (0, 0),
                         memory_space=pltpu.MemorySpace.SMEM),
        ),
        out_shape=(
            jax.ShapeDtypeStruct((N, D), jnp.float32),
            jax.ShapeDtypeStruct((1, 1), jnp.float32),
        ),
    )(*s_chunks, aA, aB, *w_chunks, *v_chunks, b3f)


def _scale_body(t, ssq, out):
    out[...] = t[...] * lax.rsqrt(ssq[0, 0])


@jax.jit
def _tc_scale(t, ssq):
    return pl.pallas_call(
        _scale_body,
        grid=(GRID,),
        in_specs=[pl.BlockSpec((BLK, D), lambda i: (i, 0)),
                  pl.BlockSpec((1, 1), lambda i: (0, 0),
                               memory_space=pltpu.MemorySpace.SMEM)],
        out_specs=pl.BlockSpec((BLK, D), lambda i: (i, 0)),
        out_shape=jax.ShapeDtypeStruct((N, D), jnp.float32),
        input_output_aliases={0: 0},
    )(t, ssq)


def _row_blocks(Wt):
    blocks = []
    for g in range(Wt.shape[0] // D):
        base = g * D
        for q in range(NCH):
            lo = base + q * DC
            hi = min(base + (q + 1) * DC, base + D)
            blk = Wt[lo:hi]
            if hi - lo < DC:
                blk = jnp.concatenate(
                    [blk, jnp.zeros((DC - (hi - lo), Wt.shape[1]),
                                    jnp.float32)], axis=0)
            blocks.append(blk)
    return jnp.concatenate(blocks, axis=0)


def _col_chunks(Wc, b):
    ws, bs = [], []
    for q in range(NCH):
        lo, hi = q * DC, min((q + 1) * DC, D)
        wq = Wc[:, lo:hi]
        bq = b[lo:hi]
        if hi - lo < DC:
            wq = jnp.concatenate(
                [wq, jnp.zeros((Wc.shape[0], DC - (hi - lo)), jnp.float32)],
                axis=1)
            bq = jnp.pad(bq, (0, DC - (hi - lo)))
        ws.append(wq)
        bs.append(bq.reshape(1, DC))
    return ws, bs


def _prep_layer_weights(W, b):
    ws, bs = _col_chunks(_row_blocks(W.T), b)
    wx = [w[:NCH * DC] for w in ws]
    wa = [w[NCH * DC:] for w in ws]
    waA = [jnp.concatenate([w[0:DC], w[2 * DC:3 * DC]], axis=0) for w in wa]
    waB = [jnp.concatenate([w[DC:2 * DC], w[3 * DC:4 * DC]], axis=0)
           for w in wa]
    return wx, waA, waB, bs


def _prep_v(W3):
    Vc = _row_blocks(W3.T)
    return [Vc[q * DC:(q + 1) * DC] for q in range(NCH)]


def kernel(features, edge_index, W1, b1, W2, b2, W3, b3):
    x = _tc_split(features)

    a0, a2 = _segsum_pass(x[0], x[2], edge_index)
    w1x, w1aA, w1aB, b1c = _prep_layer_weights(W1, b1)
    p1 = _tc_pre(x, w1x, b1c)
    a1, a3 = _segsum_pass(x[1], x[3], edge_index)
    s1 = _tc_post_partial(p1, a0, a2, w1aA)
    h = _tc_post1b(s1, a1, a3, w1aB)

    c0, c2 = _segsum_pass(h[0], h[2], edge_index)
    w2x, w2aA, w2aB, b2c = _prep_layer_weights(W2, b2)
    p2 = _tc_pre(h, w2x, b2c)
    c1, c3 = _segsum_pass(h[1], h[3], edge_index)
    s2 = _tc_post_partial(p2, c0, c2, w2aA)
    v_chunks = _prep_v(W3)
    t, ssq = _tc_finalB(s2, c1, c3, w2aB, v_chunks, b3.reshape(1, D))
    return _tc_scale(t, ssq)


# speedup vs baseline: 1.0124x; 1.0124x over previous
import functools

import jax
import jax.numpy as jnp
from jax import lax
from jax.experimental import pallas as pl
from jax.experimental.pallas import tpu as pltpu
from jax.experimental.pallas import tpu_sc as plsc

N = 10000
NPAD = 10240
E = 160000
D = 300
DC = 80
NCH = 4
NSUB = 16
CHUNK = 80
EDGES_PER_SUB = E // NSUB
NCHUNK = EDGES_PER_SUB // CHUNK
ROWS_PER_SUB = NPAD // NSUB
ZROWS = 128
BLK = 1000
GRID = N // BLK


def _sc_mesh():
    return plsc.VectorSubcoreMesh(
        core_axis_name="c", subcore_axis_name="s", num_cores=2, num_subcores=NSUB
    )


def _stage_idx(idx_all, base, cur):
    for j in range(CHUNK // 16):
        cur[pl.ds(j * 16, 16)] = idx_all[pl.ds(base + j * 16, 16)]


NBUF = 6


def _segsum_body(x0, x1, x2, x3, ei_hbm, o0, o1, o2, o3,
                 srcv, dstv, curs, rows, zbuf, acc, gsems, ssems):
    core = lax.axis_index("c")
    w = lax.axis_index("s")

    @pl.loop(0, ZROWS)
    def _(i):
        for j in range(DC // 16):
            zbuf[i, pl.ds(j * 16, 16)] = jnp.zeros((16,), jnp.float32)

    pltpu.sync_copy(ei_hbm.at[0, pl.ds(w * EDGES_PER_SUB, EDGES_PER_SUB)], srcv)
    pltpu.sync_copy(ei_hbm.at[1, pl.ds(w * EDGES_PER_SUB, EDGES_PER_SUB)], dstv)

    def run(x_hbm, out_hbm):
        for k in range(ROWS_PER_SUB // ZROWS):
            pltpu.sync_copy(zbuf,
                            acc.at[pl.ds(w * ROWS_PER_SUB + k * ZROWS, ZROWS)])
        plsc.subcore_barrier()

        def g_issue(c, b):
            pltpu.async_copy(x_hbm.at[srcv.at[pl.ds(c * CHUNK, CHUNK)]],
                             rows[b], gsems[b])

        def g_wait(c, b):
            pltpu.make_async_copy(x_hbm.at[srcv.at[pl.ds(c * CHUNK, CHUNK)]],
                                  rows[b], gsems[b]).wait()

        def s_wait(b):
            pltpu.make_async_copy(rows[b], acc.at[curs[b]], ssems[b]).wait()

        for b in range(NBUF):
            g_issue(b, b)

        tail = NCHUNK % NBUF
        main = NCHUNK - tail

        @pl.loop(0, main, step=NBUF)
        def _(i):
            for b in range(NBUF):
                g_wait(i + b, b)
                _stage_idx(dstv, (i + b) * CHUNK, curs[b])
                pltpu.async_copy(rows[b], acc.at[curs[b]], ssems[b], add=True)
            for b in range(NBUF):
                @pl.when(i + b + NBUF < NCHUNK)
                def _():
                    s_wait(b)
                    g_issue(i + b + NBUF, b)

        for b in range(tail):
            g_wait(main + b, b)
            _stage_idx(dstv, (main + b) * CHUNK, curs[b])
            pltpu.async_copy(rows[b], acc.at[curs[b]], ssems[b], add=True)
        for b in range(NBUF):
            s_wait(b)

        plsc.subcore_barrier()
        pltpu.sync_copy(acc.at[pl.ds(w * ROWS_PER_SUB, ROWS_PER_SUB)],
                        out_hbm.at[pl.ds(w * ROWS_PER_SUB, ROWS_PER_SUB)])

    @pl.when(core == 0)
    def _():
        run(x0, o0)
        run(x1, o1)

    @pl.when(core == 1)
    def _():
        run(x2, o2)
        run(x3, o3)


@jax.jit
def _segsum(x0, x1, x2, x3, edge_index):
    chunk_ty = jax.ShapeDtypeStruct((NPAD, DC), jnp.float32)
    f = pl.kernel(
        _segsum_body,
        out_type=(chunk_ty,) * NCH,
        mesh=_sc_mesh(),
        scratch_types=[
            pltpu.VMEM((EDGES_PER_SUB,), jnp.int32),
            pltpu.VMEM((EDGES_PER_SUB,), jnp.int32),
            [pltpu.VMEM((CHUNK,), jnp.int32)] * NBUF,
            [pltpu.VMEM((CHUNK, DC), jnp.float32)] * NBUF,
            pltpu.VMEM((ZROWS, DC), jnp.float32),
            pltpu.VMEM_SHARED((NPAD, DC), jnp.float32),
            [pltpu.SemaphoreType.DMA] * NBUF,
            [pltpu.SemaphoreType.DMA] * NBUF,
        ],
        compiler_params=pltpu.CompilerParams(use_tc_tiling_on_sc=False),
    )
    return f(x0, x1, x2, x3, edge_index)


def _split_body(x, o0, o1, o2, o3):
    xb = x[...]
    o0[...] = xb[:, 0:DC]
    o1[...] = xb[:, DC:2 * DC]
    o2[...] = xb[:, 2 * DC:3 * DC]
    o3[...] = jnp.concatenate(
        [xb[:, 3 * DC:D], jnp.zeros((BLK, NCH * DC - D), jnp.float32)], axis=1)


@jax.jit
def _tc_split(features):
    cspec = pl.BlockSpec((BLK, DC), lambda i: (i, 0))
    chunk_ty = jax.ShapeDtypeStruct((NPAD, DC), jnp.float32)
    return pl.pallas_call(
        _split_body,
        grid=(GRID,),
        in_specs=[pl.BlockSpec((BLK, D), lambda i: (i, 0))],
        out_specs=(cspec,) * NCH,
        out_shape=(chunk_ty,) * NCH,
    )(features)


def _pre_body(x0, x1, x2, x3, w0, w1, w2, w3, b0, b1, b2, b3,
              p0, p1, p2, p3):
    u = jnp.concatenate([x0[...], x1[...], x2[...], x3[...]], axis=1)
    for wq, bq, pq in ((w0, b0, p0), (w1, b1, p1), (w2, b2, p2), (w3, b3, p3)):
        pq[...] = jnp.dot(u, wq[...], preferred_element_type=jnp.float32) \
            + bq[...]


@jax.jit
def _tc_pre(x_chunks, wx_chunks, b_chunks):
    cspec = pl.BlockSpec((BLK, DC), lambda i: (i, 0))
    wspec = pl.BlockSpec((4 * DC, DC), lambda i: (0, 0))
    bspec = pl.BlockSpec((1, DC), lambda i: (0, 0))
    chunk_ty = jax.ShapeDtypeStruct((NPAD, DC), jnp.float32)
    return pl.pallas_call(
        _pre_body,
        grid=(GRID,),
        in_specs=[cspec] * 4 + [wspec] * 4 + [bspec] * 4,
        out_specs=(cspec,) * 4,
        out_shape=(chunk_ty,) * 4,
    )(*x_chunks, *wx_chunks, *b_chunks)


def _post1_body(p0, p1, p2, p3, a0, a1, a2, a3, w0, w1, w2, w3,
                h0, h1, h2, h3):
    u = jnp.concatenate([a0[...], a1[...], a2[...], a3[...]], axis=1)
    for wq, pq, hq in ((w0, p0, h0), (w1, p1, h1), (w2, p2, h2), (w3, p3, h3)):
        z = pq[...] + jnp.dot(u, wq[...], preferred_element_type=jnp.float32)
        hq[...] = jnp.where(z > 0, z, 0.01 * z)


@jax.jit
def _tc_post1(p_chunks, a_chunks, wa_chunks):
    cspec = pl.BlockSpec((BLK, DC), lambda i: (i, 0))
    wspec = pl.BlockSpec((4 * DC, DC), lambda i: (0, 0))
    chunk_ty = jax.ShapeDtypeStruct((NPAD, DC), jnp.float32)
    return pl.pallas_call(
        _post1_body,
        grid=(GRID,),
        in_specs=[cspec] * 8 + [wspec] * 4,
        out_specs=(cspec,) * 4,
        out_shape=(chunk_ty,) * 4,
    )(*p_chunks, *a_chunks, *wa_chunks)


def _final_body(p0, p1, p2, p3, a0, a1, a2, a3, w0, w1, w2, w3,
                v0, v1, v2, v3, b3f, t_out, ssq):
    i = pl.program_id(0)
    u = jnp.concatenate([a0[...], a1[...], a2[...], a3[...]], axis=1)
    acc = b3f[...]
    for wq, pq, vq in ((w0, p0, v0), (w1, p1, v1), (w2, p2, v2), (w3, p3, v3)):
        z = pq[...] + jnp.dot(u, wq[...], preferred_element_type=jnp.float32)
        acc = acc + jnp.dot(z, vq[...], preferred_element_type=jnp.float32)
    t = jnp.tanh(acc)
    t_out[...] = t

    @pl.when(i == 0)
    def _():
        ssq[0, 0] = 0.0

    ssq[0, 0] += jnp.sum(t * t)


@jax.jit
def _tc_final(p_chunks, a_chunks, wa_chunks, v_chunks, b3f):
    cspec = pl.BlockSpec((BLK, DC), lambda i: (i, 0))
    wspec = pl.BlockSpec((4 * DC, DC), lambda i: (0, 0))
    vspec = pl.BlockSpec((DC, D), lambda i: (0, 0))
    b3spec = pl.BlockSpec((1, D), lambda i: (0, 0))
    return pl.pallas_call(
        _final_body,
        grid=(GRID,),
        in_specs=[cspec] * 8 + [wspec] * 4 + [vspec] * 4 + [b3spec],
        out_specs=(
            pl.BlockSpec((BLK, D), lambda i: (i, 0)),
            pl.BlockSpec((1, 1), lambda i: (0, 0),
                         memory_space=pltpu.MemorySpace.SMEM),
        ),
        out_shape=(
            jax.ShapeDtypeStruct((N, D), jnp.float32),
            jax.ShapeDtypeStruct((1, 1), jnp.float32),
        ),
    )(*p_chunks, *a_chunks, *wa_chunks, *v_chunks, b3f)


def _scale_body(t, ssq, out):
    out[...] = t[...] * lax.rsqrt(ssq[0, 0])


@jax.jit
def _tc_scale(t, ssq):
    return pl.pallas_call(
        _scale_body,
        grid=(GRID,),
        in_specs=[pl.BlockSpec((BLK, D), lambda i: (i, 0)),
                  pl.BlockSpec((1, 1), lambda i: (0, 0),
                               memory_space=pltpu.MemorySpace.SMEM)],
        out_specs=pl.BlockSpec((BLK, D), lambda i: (i, 0)),
        out_shape=jax.ShapeDtypeStruct((N, D), jnp.float32),
        input_output_aliases={0: 0},
    )(t, ssq)


def _row_blocks(Wt):
    blocks = []
    for g in range(Wt.shape[0] // D):
        base = g * D
        for q in range(NCH):
            lo = base + q * DC
            hi = min(base + (q + 1) * DC, base + D)
            blk = Wt[lo:hi]
            if hi - lo < DC:
                blk = jnp.concatenate(
                    [blk, jnp.zeros((DC - (hi - lo), Wt.shape[1]),
                                    jnp.float32)], axis=0)
            blocks.append(blk)
    return jnp.concatenate(blocks, axis=0)


def _col_chunks(Wc, b):
    ws, bs = [], []
    for q in range(NCH):
        lo, hi = q * DC, min((q + 1) * DC, D)
        wq = Wc[:, lo:hi]
        bq = b[lo:hi]
        if hi - lo < DC:
            wq = jnp.concatenate(
                [wq, jnp.zeros((Wc.shape[0], DC - (hi - lo)), jnp.float32)],
                axis=1)
            bq = jnp.pad(bq, (0, DC - (hi - lo)))
        ws.append(wq)
        bs.append(bq.reshape(1, DC))
    return ws, bs


def _prep_layer_weights(W, b):
    ws, bs = _col_chunks(_row_blocks(W.T), b)
    wx = [w[:NCH * DC] for w in ws]
    wa = [w[NCH * DC:] for w in ws]
    return wx, wa, bs


def _prep_v(W3):
    Vc = _row_blocks(W3.T)
    return [Vc[q * DC:(q + 1) * DC] for q in range(NCH)]


def kernel(features, edge_index, W1, b1, W2, b2, W3, b3):
    x_chunks = _tc_split(features)

    a1 = _segsum(*x_chunks, edge_index)
    w1x, w1a, b1c = _prep_layer_weights(W1, b1)
    p1 = _tc_pre(x_chunks, w1x, b1c)
    h = _tc_post1(p1, a1, w1a)

    a2 = _segsum(*h, edge_index)
    w2x, w2a, b2c = _prep_layer_weights(W2, b2)
    p2 = _tc_pre(h, w2x, b2c)
    v_chunks = _prep_v(W3)
    t, ssq = _tc_final(p2, a2, w2a, v_chunks, b3.reshape(1, D))
    return _tc_scale(t, ssq)


# speedup vs baseline: 1.0227x; 1.0102x over previous
import functools

import jax
import jax.numpy as jnp
from jax import lax
from jax.experimental import pallas as pl
from jax.experimental.pallas import tpu as pltpu
from jax.experimental.pallas import tpu_sc as plsc

N = 10000
NPAD = 10240
E = 160000
D = 300
DC = 80
NCH = 4
NSUB = 16
CHUNK = 80
EDGES_PER_SUB = E // NSUB
NCHUNK = EDGES_PER_SUB // CHUNK
ROWS_PER_SUB = NPAD // NSUB
ZROWS = 128
BLK = 2000
GRID = N // BLK


def _sc_mesh():
    return plsc.VectorSubcoreMesh(
        core_axis_name="c", subcore_axis_name="s", num_cores=2, num_subcores=NSUB
    )


def _stage_idx(idx_all, base, cur):
    for j in range(CHUNK // 16):
        cur[pl.ds(j * 16, 16)] = idx_all[pl.ds(base + j * 16, 16)]


NBUF = 6


def _segsum_body(x0, x1, x2, x3, ei_hbm, o0, o1, o2, o3,
                 srcv, dstv, curs, rows, zbuf, acc, gsems, ssems):
    core = lax.axis_index("c")
    w = lax.axis_index("s")

    @pl.loop(0, ZROWS)
    def _(i):
        for j in range(DC // 16):
            zbuf[i, pl.ds(j * 16, 16)] = jnp.zeros((16,), jnp.float32)

    pltpu.sync_copy(ei_hbm.at[0, pl.ds(w * EDGES_PER_SUB, EDGES_PER_SUB)], srcv)
    pltpu.sync_copy(ei_hbm.at[1, pl.ds(w * EDGES_PER_SUB, EDGES_PER_SUB)], dstv)

    def run(x_hbm, out_hbm):
        for k in range(ROWS_PER_SUB // ZROWS):
            pltpu.sync_copy(zbuf,
                            acc.at[pl.ds(w * ROWS_PER_SUB + k * ZROWS, ZROWS)])
        plsc.subcore_barrier()

        def g_issue(c, b):
            pltpu.async_copy(x_hbm.at[srcv.at[pl.ds(c * CHUNK, CHUNK)]],
                             rows[b], gsems[b])

        def g_wait(c, b):
            pltpu.make_async_copy(x_hbm.at[srcv.at[pl.ds(c * CHUNK, CHUNK)]],
                                  rows[b], gsems[b]).wait()

        def s_wait(b):
            pltpu.make_async_copy(rows[b], acc.at[curs[b]], ssems[b]).wait()

        for b in range(NBUF):
            g_issue(b, b)

        tail = NCHUNK % NBUF
        main = NCHUNK - tail

        @pl.loop(0, main, step=NBUF)
        def _(i):
            for b in range(NBUF):
                g_wait(i + b, b)
                _stage_idx(dstv, (i + b) * CHUNK, curs[b])
                pltpu.async_copy(rows[b], acc.at[curs[b]], ssems[b], add=True)
            for b in range(NBUF):
                @pl.when(i + b + NBUF < NCHUNK)
                def _():
                    s_wait(b)
                    g_issue(i + b + NBUF, b)

        for b in range(tail):
            g_wait(main + b, b)
            _stage_idx(dstv, (main + b) * CHUNK, curs[b])
            pltpu.async_copy(rows[b], acc.at[curs[b]], ssems[b], add=True)
        for b in range(NBUF):
            s_wait(b)

        plsc.subcore_barrier()
        pltpu.sync_copy(acc.at[pl.ds(w * ROWS_PER_SUB, ROWS_PER_SUB)],
                        out_hbm.at[pl.ds(w * ROWS_PER_SUB, ROWS_PER_SUB)])

    @pl.when(core == 0)
    def _():
        run(x0, o0)
        run(x1, o1)

    @pl.when(core == 1)
    def _():
        run(x2, o2)
        run(x3, o3)


@jax.jit
def _segsum(x0, x1, x2, x3, edge_index):
    chunk_ty = jax.ShapeDtypeStruct((NPAD, DC), jnp.float32)
    f = pl.kernel(
        _segsum_body,
        out_type=(chunk_ty,) * NCH,
        mesh=_sc_mesh(),
        scratch_types=[
            pltpu.VMEM((EDGES_PER_SUB,), jnp.int32),
            pltpu.VMEM((EDGES_PER_SUB,), jnp.int32),
            [pltpu.VMEM((CHUNK,), jnp.int32)] * NBUF,
            [pltpu.VMEM((CHUNK, DC), jnp.float32)] * NBUF,
            pltpu.VMEM((ZROWS, DC), jnp.float32),
            pltpu.VMEM_SHARED((NPAD, DC), jnp.float32),
            [pltpu.SemaphoreType.DMA] * NBUF,
            [pltpu.SemaphoreType.DMA] * NBUF,
        ],
        compiler_params=pltpu.CompilerParams(use_tc_tiling_on_sc=False),
    )
    return f(x0, x1, x2, x3, edge_index)


def _split_body(x, o0, o1, o2, o3):
    xb = x[...]
    o0[...] = xb[:, 0:DC]
    o1[...] = xb[:, DC:2 * DC]
    o2[...] = xb[:, 2 * DC:3 * DC]
    o3[...] = jnp.concatenate(
        [xb[:, 3 * DC:D], jnp.zeros((BLK, NCH * DC - D), jnp.float32)], axis=1)


@jax.jit
def _tc_split(features):
    cspec = pl.BlockSpec((BLK, DC), lambda i: (i, 0))
    chunk_ty = jax.ShapeDtypeStruct((NPAD, DC), jnp.float32)
    return pl.pallas_call(
        _split_body,
        grid=(GRID,),
        in_specs=[pl.BlockSpec((BLK, D), lambda i: (i, 0))],
        out_specs=(cspec,) * NCH,
        out_shape=(chunk_ty,) * NCH,
    )(features)


def _pre_body(x0, x1, x2, x3, w0, w1, w2, w3, b0, b1, b2, b3,
              p0, p1, p2, p3):
    u = jnp.concatenate([x0[...], x1[...], x2[...], x3[...]], axis=1)
    for wq, bq, pq in ((w0, b0, p0), (w1, b1, p1), (w2, b2, p2), (w3, b3, p3)):
        pq[...] = jnp.dot(u, wq[...], preferred_element_type=jnp.float32) \
            + bq[...]


@jax.jit
def _tc_pre(x_chunks, wx_chunks, b_chunks):
    cspec = pl.BlockSpec((BLK, DC), lambda i: (i, 0))
    wspec = pl.BlockSpec((4 * DC, DC), lambda i: (0, 0))
    bspec = pl.BlockSpec((1, DC), lambda i: (0, 0))
    chunk_ty = jax.ShapeDtypeStruct((NPAD, DC), jnp.float32)
    return pl.pallas_call(
        _pre_body,
        grid=(GRID,),
        in_specs=[cspec] * 4 + [wspec] * 4 + [bspec] * 4,
        out_specs=(cspec,) * 4,
        out_shape=(chunk_ty,) * 4,
    )(*x_chunks, *wx_chunks, *b_chunks)


def _post1_body(p0, p1, p2, p3, a0, a1, a2, a3, w0, w1, w2, w3,
                h0, h1, h2, h3):
    u = jnp.concatenate([a0[...], a1[...], a2[...], a3[...]], axis=1)
    for wq, pq, hq in ((w0, p0, h0), (w1, p1, h1), (w2, p2, h2), (w3, p3, h3)):
        z = pq[...] + jnp.dot(u, wq[...], preferred_element_type=jnp.float32)
        hq[...] = jnp.where(z > 0, z, 0.01 * z)


@jax.jit
def _tc_post1(p_chunks, a_chunks, wa_chunks):
    cspec = pl.BlockSpec((BLK, DC), lambda i: (i, 0))
    wspec = pl.BlockSpec((4 * DC, DC), lambda i: (0, 0))
    chunk_ty = jax.ShapeDtypeStruct((NPAD, DC), jnp.float32)
    return pl.pallas_call(
        _post1_body,
        grid=(GRID,),
        in_specs=[cspec] * 8 + [wspec] * 4,
        out_specs=(cspec,) * 4,
        out_shape=(chunk_ty,) * 4,
    )(*p_chunks, *a_chunks, *wa_chunks)


def _final_body(p0, p1, p2, p3, a0, a1, a2, a3, w0, w1, w2, w3,
                v0, v1, v2, v3, b3f, t_out, ssq):
    i = pl.program_id(0)
    u = jnp.concatenate([a0[...], a1[...], a2[...], a3[...]], axis=1)
    acc = b3f[...]
    for wq, pq, vq in ((w0, p0, v0), (w1, p1, v1), (w2, p2, v2), (w3, p3, v3)):
        z = pq[...] + jnp.dot(u, wq[...], preferred_element_type=jnp.float32)
        acc = acc + jnp.dot(z, vq[...], preferred_element_type=jnp.float32)
    t = jnp.tanh(acc)
    t_out[...] = t

    @pl.when(i == 0)
    def _():
        ssq[0, 0] = 0.0

    ssq[0, 0] += jnp.sum(t * t)


@jax.jit
def _tc_final(p_chunks, a_chunks, wa_chunks, v_chunks, b3f):
    cspec = pl.BlockSpec((BLK, DC), lambda i: (i, 0))
    wspec = pl.BlockSpec((4 * DC, DC), lambda i: (0, 0))
    vspec = pl.BlockSpec((DC, D), lambda i: (0, 0))
    b3spec = pl.BlockSpec((1, D), lambda i: (0, 0))
    return pl.pallas_call(
        _final_body,
        grid=(GRID,),
        in_specs=[cspec] * 8 + [wspec] * 4 + [vspec] * 4 + [b3spec],
        out_specs=(
            pl.BlockSpec((BLK, D), lambda i: (i, 0)),
            pl.BlockSpec((1, 1), lambda i: (0, 0),
                         memory_space=pltpu.MemorySpace.SMEM),
        ),
        out_shape=(
            jax.ShapeDtypeStruct((N, D), jnp.float32),
            jax.ShapeDtypeStruct((1, 1), jnp.float32),
        ),
    )(*p_chunks, *a_chunks, *wa_chunks, *v_chunks, b3f)


def _scale_body(t, ssq, out):
    out[...] = t[...] * lax.rsqrt(ssq[0, 0])


@jax.jit
def _tc_scale(t, ssq):
    return pl.pallas_call(
        _scale_body,
        grid=(GRID,),
        in_specs=[pl.BlockSpec((BLK, D), lambda i: (i, 0)),
                  pl.BlockSpec((1, 1), lambda i: (0, 0),
                               memory_space=pltpu.MemorySpace.SMEM)],
        out_specs=pl.BlockSpec((BLK, D), lambda i: (i, 0)),
        out_shape=jax.ShapeDtypeStruct((N, D), jnp.float32),
        input_output_aliases={0: 0},
    )(t, ssq)


def _row_blocks(Wt):
    blocks = []
    for g in range(Wt.shape[0] // D):
        base = g * D
        for q in range(NCH):
            lo = base + q * DC
            hi = min(base + (q + 1) * DC, base + D)
            blk = Wt[lo:hi]
            if hi - lo < DC:
                blk = jnp.concatenate(
                    [blk, jnp.zeros((DC - (hi - lo), Wt.shape[1]),
                                    jnp.float32)], axis=0)
            blocks.append(blk)
    return jnp.concatenate(blocks, axis=0)


def _col_chunks(Wc, b):
    ws, bs = [], []
    for q in range(NCH):
        lo, hi = q * DC, min((q + 1) * DC, D)
        wq = Wc[:, lo:hi]
        bq = b[lo:hi]
        if hi - lo < DC:
            wq = jnp.concatenate(
                [wq, jnp.zeros((Wc.shape[0], DC - (hi - lo)), jnp.float32)],
                axis=1)
            bq = jnp.pad(bq, (0, DC - (hi - lo)))
        ws.append(wq)
        bs.append(bq.reshape(1, DC))
    return ws, bs


def _prep_layer_weights(W, b):
    ws, bs = _col_chunks(_row_blocks(W.T), b)
    wx = [w[:NCH * DC] for w in ws]
    wa = [w[NCH * DC:] for w in ws]
    return wx, wa, bs


def _prep_v(W3):
    Vc = _row_blocks(W3.T)
    return [Vc[q * DC:(q + 1) * DC] for q in range(NCH)]


def kernel(features, edge_index, W1, b1, W2, b2, W3, b3):
    x_chunks = _tc_split(features)

    a1 = _segsum(*x_chunks, edge_index)
    w1x, w1a, b1c = _prep_layer_weights(W1, b1)
    p1 = _tc_pre(x_chunks, w1x, b1c)
    h = _tc_post1(p1, a1, w1a)

    a2 = _segsum(*h, edge_index)
    w2x, w2a, b2c = _prep_layer_weights(W2, b2)
    p2 = _tc_pre(h, w2x, b2c)
    v_chunks = _prep_v(W3)
    t, ssq = _tc_final(p2, a2, w2a, v_chunks, b3.reshape(1, D))
    return _tc_scale(t, ssq)
